# Initial kernel scaffold; baseline (speedup 1.0000x reference)
import jax
import jax.numpy as jnp
from jax.experimental import pallas as pl


def kernel(node_attr, edge_index, edge_attr, global_attr, W1, b1, W2, b2):
    raise NotImplementedError("write your pallas kernel here")


# speedup vs baseline: 2.3360x; 2.3360x over previous
import functools

import jax
import jax.numpy as jnp
from jax import lax
from jax.experimental import pallas as pl
from jax.experimental.pallas import tpu as pltpu
from jax.experimental.pallas import tpu_sc as plsc

N_NODES = 10000
N_EDGES = 320000
D_FEAT = 128
D_EDGE = 16
D_GLOBAL = 32
LATENT = 32
D_OUT = 128

_NC = 2
_NS = 16
_NW = _NC * _NS
_EPW = N_EDGES // _NW
_CH = 80
_NCHUNK = _EPW // _CH


def _proj_body(node_ref, w_ref, outs_ref, outr_ref):
    t = jnp.dot(node_ref[...], w_ref[...], preferred_element_type=jnp.float32)
    outs_ref[...] = t[:, :LATENT]
    outr_ref[...] = t[:, LATENT:]


def _node_projections(node_attr, w_sr):
    return pl.pallas_call(
        _proj_body,
        out_shape=[
            jax.ShapeDtypeStruct((N_NODES, LATENT), jnp.float32),
            jax.ShapeDtypeStruct((N_NODES, LATENT), jnp.float32),
        ],
    )(node_attr, w_sr)


def _gather_body(tabs_hbm, tabr_hbm, send_hbm, recv_hbm, outs_hbm, outr_hbm,
                 idx_v, rows_v, idx2_v, rows2_v, sem, sem2):
    wid = lax.axis_index("s") * _NC + lax.axis_index("c")
    base = pl.multiple_of(wid * _EPW, 8)

    def chunk(j, carry):
        s0 = pl.multiple_of(base + j * _CH, 8)
        pltpu.sync_copy(send_hbm.at[pl.ds(s0, _CH)], idx_v)
        pltpu.async_copy(tabs_hbm.at[idx_v], rows_v, sem).wait()
        pltpu.sync_copy(rows_v, outs_hbm.at[pl.ds(s0, _CH)])
        pltpu.sync_copy(recv_hbm.at[pl.ds(s0, _CH)], idx2_v)
        pltpu.async_copy(tabr_hbm.at[idx2_v], rows2_v, sem2).wait()
        pltpu.sync_copy(rows2_v, outr_hbm.at[pl.ds(s0, _CH)])
        return carry

    lax.fori_loop(0, _NCHUNK, chunk, 0)


def _gather_projections(tabs, tabr, senders, receivers):
    mesh = plsc.VectorSubcoreMesh(core_axis_name="c", subcore_axis_name="s")
    k = functools.partial(
        pl.kernel,
        out_type=[
            jax.ShapeDtypeStruct((N_EDGES, LATENT), jnp.float32),
            jax.ShapeDtypeStruct((N_EDGES, LATENT), jnp.float32),
        ],
        mesh=mesh,
        scratch_types=[
            pltpu.VMEM((_CH,), jnp.int32),
            pltpu.VMEM((_CH, LATENT), jnp.float32),
            pltpu.VMEM((_CH,), jnp.int32),
            pltpu.VMEM((_CH, LATENT), jnp.float32),
            pltpu.SemaphoreType.DMA,
            pltpu.SemaphoreType.DMA,
        ],
        compiler_params=pltpu.CompilerParams(use_tc_tiling_on_sc=False),
    )(_gather_body)
    return k(tabs, tabr, senders, receivers)


_BE = 6400


def _mlp_body(e_ref, gs_ref, gr_ref, w1e_ref, w1g_ref, g_ref, b1_ref,
              w2_ref, b2_ref, out_ref):
    bias = b1_ref[...] + jnp.dot(g_ref[...], w1g_ref[...],
                                 preferred_element_type=jnp.float32)
    pre = (gs_ref[...] + gr_ref[...]
           + jnp.dot(e_ref[...], w1e_ref[...],
                     preferred_element_type=jnp.float32)
           + bias)
    h = jnp.maximum(pre, 0.0)
    out_ref[...] = jnp.dot(h, w2_ref[...],
                           preferred_element_type=jnp.float32) + b2_ref[...]


def _edge_mlp(edge_attr, gs, gr, w1e, w1g, g, b1, w2, b2):
    nblk = N_EDGES // _BE
    full = lambda shape: pl.BlockSpec(shape, lambda i: (0, 0))
    return pl.pallas_call(
        _mlp_body,
        grid=(nblk,),
        in_specs=[
            pl.BlockSpec((_BE, D_EDGE), lambda i: (i, 0)),
            pl.BlockSpec((_BE, LATENT), lambda i: (i, 0)),
            pl.BlockSpec((_BE, LATENT), lambda i: (i, 0)),
            full((D_EDGE, LATENT)),
            full((D_GLOBAL, LATENT)),
            full((1, D_GLOBAL)),
            full((1, LATENT)),
            full((LATENT, D_OUT)),
            full((1, D_OUT)),
        ],
        out_specs=pl.BlockSpec((_BE, D_OUT), lambda i: (i, 0)),
        out_shape=jax.ShapeDtypeStruct((N_EDGES, D_OUT), jnp.float32),
    )(edge_attr, gs, gr, w1e, w1g, g, b1, w2, b2)


def kernel(node_attr, edge_index, edge_attr, global_attr, W1, b1, W2, b2):
    senders = edge_index[0].astype(jnp.int32)
    receivers = edge_index[1].astype(jnp.int32)
    w1e = W1[:D_EDGE]
    w_sr = W1[D_EDGE:D_EDGE + 2 * D_FEAT]
    w_sr = jnp.concatenate(
        [w_sr[:D_FEAT], w_sr[D_FEAT:]], axis=1)
    w1g = W1[D_EDGE + 2 * D_FEAT:]
    tabs, tabr = _node_projections(node_attr, w_sr)
    gs, gr = _gather_projections(tabs, tabr, senders, receivers)
    return _edge_mlp(edge_attr, gs, gr, w1e, w1g, global_attr,
                     b1.reshape(1, LATENT), W2, b2.reshape(1, D_OUT))


# speedup vs baseline: 4.0595x; 1.7378x over previous
import functools

import jax
import jax.numpy as jnp
from jax import lax
from jax.experimental import pallas as pl
from jax.experimental.pallas import tpu as pltpu
from jax.experimental.pallas import tpu_sc as plsc

N_NODES = 10000
N_EDGES = 320000
D_FEAT = 128
D_EDGE = 16
D_GLOBAL = 32
LATENT = 32
D_OUT = 128

_NC = 2
_NS = 16
_NW = _NC * _NS
_EPW = N_EDGES // _NW
_CH = 80
_NCHUNK = _EPW // _CH


def _proj_body(node_ref, w_ref, outs_ref, outr_ref):
    t = jnp.dot(node_ref[...], w_ref[...], preferred_element_type=jnp.float32)
    outs_ref[...] = t[:, :LATENT]
    outr_ref[...] = t[:, LATENT:]


def _node_projections(node_attr, w_sr):
    return pl.pallas_call(
        _proj_body,
        out_shape=[
            jax.ShapeDtypeStruct((N_NODES, LATENT), jnp.float32),
            jax.ShapeDtypeStruct((N_NODES, LATENT), jnp.float32),
        ],
    )(node_attr, w_sr)


_GROWS = _CH * LATENT // 128
_WROWS = _EPW * LATENT // 128


def _gather_body(tabs_hbm, tabr_hbm, eidx_hbm, out_hbm,
                 idxs_v, idxr_v, rows_s, rows_r, sum_v, sems, semr):
    wid = lax.axis_index("s") * _NC + lax.axis_index("c")
    base = pl.multiple_of(wid * _EPW, 8)
    pltpu.sync_copy(eidx_hbm.at[0, pl.ds(base, _EPW)], idxs_v)
    pltpu.sync_copy(eidx_hbm.at[1, pl.ds(base, _EPW)], idxr_v)

    def chunk(j, carry):
        cs = pl.ds(pl.multiple_of(j * _CH, 8), _CH)
        cps = pltpu.async_copy(tabs_hbm.at[idxs_v.at[cs]], rows_s, sems)
        cpr = pltpu.async_copy(tabr_hbm.at[idxr_v.at[cs]], rows_r, semr)
        cps.wait()
        cpr.wait()
        for i in range(_CH * LATENT // 16):
            r, c = divmod(i, 2)
            q, p = divmod(i, 8)
            sum_v[q, pl.ds(p * 16, 16)] = (
                rows_s[r, pl.ds(c * 16, 16)] + rows_r[r, pl.ds(c * 16, 16)])
        row0 = wid * _WROWS + j * _GROWS
        pltpu.sync_copy(sum_v, out_hbm.at[pl.ds(row0, _GROWS)])
        return carry

    lax.fori_loop(0, _NCHUNK, chunk, 0)


def _gather_projections(tabs, tabr, edge_index):
    mesh = plsc.VectorSubcoreMesh(core_axis_name="c", subcore_axis_name="s")
    k = functools.partial(
        pl.kernel,
        out_type=jax.ShapeDtypeStruct((N_EDGES * LATENT // 128, 128),
                                      jnp.float32),
        mesh=mesh,
        scratch_types=[
            pltpu.VMEM((_EPW,), jnp.int32),
            pltpu.VMEM((_EPW,), jnp.int32),
            pltpu.VMEM((_CH, LATENT), jnp.float32),
            pltpu.VMEM((_CH, LATENT), jnp.float32),
            pltpu.VMEM((_GROWS, 128), jnp.float32),
            pltpu.SemaphoreType.DMA,
            pltpu.SemaphoreType.DMA,
        ],
        compiler_params=pltpu.CompilerParams(use_tc_tiling_on_sc=False),
    )(_gather_body)
    return k(tabs, tabr, edge_index)


_BE = 6400


def _mlp_body(e_ref, gsum_ref, w1e_ref, w1g_ref, g_ref, b1_ref,
              w2_ref, b2_ref, out_ref):
    bias = b1_ref[...] + jnp.dot(g_ref[...], w1g_ref[...],
                                 preferred_element_type=jnp.float32)
    gsum = jnp.concatenate(
        [gsum_ref[:, k * LATENT:(k + 1) * LATENT] for k in range(4)], axis=0)
    pre = (gsum
           + jnp.dot(e_ref[...], w1e_ref[...],
                     preferred_element_type=jnp.float32)
           + bias)
    h = jnp.maximum(pre, 0.0)
    out_ref[...] = jnp.dot(h, w2_ref[...],
                           preferred_element_type=jnp.float32) + b2_ref[...]


def _edge_mlp(edge_attr, gsum, w1e, w1g, g, b1, w2, b2):
    nblk = N_EDGES // _BE
    full = lambda shape: pl.BlockSpec(shape, lambda i: (0, 0))
    return pl.pallas_call(
        _mlp_body,
        grid=(nblk,),
        in_specs=[
            pl.BlockSpec((_BE, D_EDGE), lambda i: (i, 0)),
            pl.BlockSpec((_BE * LATENT // 128, 128), lambda i: (i, 0)),
            full((D_EDGE, LATENT)),
            full((D_GLOBAL, LATENT)),
            full((1, D_GLOBAL)),
            full((1, LATENT)),
            full((LATENT, D_OUT)),
            full((1, D_OUT)),
        ],
        out_specs=pl.BlockSpec((_BE, D_OUT), lambda i: (i, 0)),
        out_shape=jax.ShapeDtypeStruct((N_EDGES, D_OUT), jnp.float32),
    )(edge_attr, gsum, w1e, w1g, g, b1, w2, b2)


def kernel(node_attr, edge_index, edge_attr, global_attr, W1, b1, W2, b2):
    eidx = edge_index.astype(jnp.int32)
    nblk = N_EDGES // _BE
    eidx = (eidx.reshape(2, nblk, 4, _BE // 4)
            .swapaxes(2, 3).reshape(2, N_EDGES))
    w1e = W1[:D_EDGE]
    w_sr = W1[D_EDGE:D_EDGE + 2 * D_FEAT]
    w_sr = jnp.concatenate(
        [w_sr[:D_FEAT], w_sr[D_FEAT:]], axis=1)
    w1g = W1[D_EDGE + 2 * D_FEAT:]
    tabs, tabr = _node_projections(node_attr, w_sr)
    gsum = _gather_projections(tabs, tabr, eidx)
    return _edge_mlp(edge_attr, gsum, w1e, w1g, global_attr,
                     b1.reshape(1, LATENT), W2, b2.reshape(1, D_OUT))


# speedup vs baseline: 5.2513x; 1.2936x over previous
import functools

import jax
import jax.numpy as jnp
from jax import lax
from jax.experimental import pallas as pl
from jax.experimental.pallas import tpu as pltpu
from jax.experimental.pallas import tpu_sc as plsc

N_NODES = 10000
N_EDGES = 320000
D_FEAT = 128
D_EDGE = 16
D_GLOBAL = 32
LATENT = 32
D_OUT = 128

_NC = 2
_NS = 16
_NW = _NC * _NS
_EPW = N_EDGES // _NW
_CH = 80
_NCHUNK = _EPW // _CH


def _proj_body(node_ref, w_ref, outs_ref, outr_ref):
    t = jnp.dot(node_ref[...], w_ref[...], preferred_element_type=jnp.float32)
    outs_ref[...] = t[:, :LATENT]
    outr_ref[...] = t[:, LATENT:]


def _node_projections(node_attr, w_sr):
    return pl.pallas_call(
        _proj_body,
        out_shape=[
            jax.ShapeDtypeStruct((N_NODES, LATENT), jnp.float32),
            jax.ShapeDtypeStruct((N_NODES, LATENT), jnp.float32),
        ],
    )(node_attr, w_sr)


_GROWS = _CH * LATENT // 128
_WROWS = _EPW * LATENT // 128


def _gather_body(tabs_hbm, tabr_hbm, eidx_hbm, out_hbm,
                 idxs_v, idxr_v, rows_s, rows_r, sum_v, sems, semr):
    wid = lax.axis_index("s") * _NC + lax.axis_index("c")
    base = pl.multiple_of(wid * _EPW, 8)
    pltpu.sync_copy(eidx_hbm.at[0, pl.ds(base, _EPW)], idxs_v)
    pltpu.sync_copy(eidx_hbm.at[1, pl.ds(base, _EPW)], idxr_v)

    def chunk(j, carry):
        cs = pl.ds(pl.multiple_of(j * _CH, 8), _CH)
        cps = pltpu.async_copy(tabs_hbm.at[idxs_v.at[cs]], rows_s, sems)
        cpr = pltpu.async_copy(tabr_hbm.at[idxr_v.at[cs]], rows_r, semr)
        cps.wait()
        cpr.wait()
        for i in range(_CH * LATENT // 16):
            r, c = divmod(i, 2)
            sum_v[r, pl.ds(c * 16, 16)] = (
                rows_s[r, pl.ds(c * 16, 16)] + rows_r[r, pl.ds(c * 16, 16)])
        e0 = base + j * _CH
        blk = e0 // _BE
        rem = e0 - blk * _BE
        kk = rem // (_BE // 4)
        r0 = rem - kk * (_BE // 4)
        dst = out_hbm.at[pl.ds((_BE // 4) * blk + r0, _CH),
                         pl.ds(LATENT * kk, LATENT)]
        pltpu.sync_copy(sum_v, dst)
        return carry

    lax.fori_loop(0, _NCHUNK, chunk, 0)


def _gather_projections(tabs, tabr, edge_index):
    mesh = plsc.VectorSubcoreMesh(core_axis_name="c", subcore_axis_name="s")
    k = functools.partial(
        pl.kernel,
        out_type=jax.ShapeDtypeStruct((N_EDGES * LATENT // 128, 128),
                                      jnp.float32),
        mesh=mesh,
        scratch_types=[
            pltpu.VMEM((_EPW,), jnp.int32),
            pltpu.VMEM((_EPW,), jnp.int32),
            pltpu.VMEM((_CH, LATENT), jnp.float32),
            pltpu.VMEM((_CH, LATENT), jnp.float32),
            pltpu.VMEM((_CH, LATENT), jnp.float32),
            pltpu.SemaphoreType.DMA,
            pltpu.SemaphoreType.DMA,
        ],
        compiler_params=pltpu.CompilerParams(use_tc_tiling_on_sc=False),
    )(_gather_body)
    return k(tabs, tabr, edge_index)


_BE = 6400


def _mlp_body(e_ref, gsum_ref, w1e_ref, w1g_ref, g_ref, b1_ref,
              w2_ref, b2_ref, out_ref):
    bias = b1_ref[...] + jnp.dot(g_ref[...], w1g_ref[...],
                                 preferred_element_type=jnp.float32)
    gsum = jnp.concatenate(
        [gsum_ref[:, k * LATENT:(k + 1) * LATENT] for k in range(4)], axis=0)
    pre = (gsum
           + jnp.dot(e_ref[...], w1e_ref[...],
                     preferred_element_type=jnp.float32)
           + bias)
    h = jnp.maximum(pre, 0.0)
    out_ref[...] = jnp.dot(h, w2_ref[...],
                           preferred_element_type=jnp.float32) + b2_ref[...]


def _edge_mlp(edge_attr, gsum, w1e, w1g, g, b1, w2, b2):
    nblk = N_EDGES // _BE
    full = lambda shape: pl.BlockSpec(shape, lambda i: (0, 0))
    return pl.pallas_call(
        _mlp_body,
        grid=(nblk,),
        in_specs=[
            pl.BlockSpec((_BE, D_EDGE), lambda i: (i, 0)),
            pl.BlockSpec((_BE * LATENT // 128, 128), lambda i: (i, 0)),
            full((D_EDGE, LATENT)),
            full((D_GLOBAL, LATENT)),
            full((1, D_GLOBAL)),
            full((1, LATENT)),
            full((LATENT, D_OUT)),
            full((1, D_OUT)),
        ],
        out_specs=pl.BlockSpec((_BE, D_OUT), lambda i: (i, 0)),
        out_shape=jax.ShapeDtypeStruct((N_EDGES, D_OUT), jnp.float32),
    )(edge_attr, gsum, w1e, w1g, g, b1, w2, b2)


def kernel(node_attr, edge_index, edge_attr, global_attr, W1, b1, W2, b2):
    eidx = edge_index.astype(jnp.int32)
    w1e = W1[:D_EDGE]
    w_sr = W1[D_EDGE:D_EDGE + 2 * D_FEAT]
    w_sr = jnp.concatenate(
        [w_sr[:D_FEAT], w_sr[D_FEAT:]], axis=1)
    w1g = W1[D_EDGE + 2 * D_FEAT:]
    tabs, tabr = _node_projections(node_attr, w_sr)
    gsum = _gather_projections(tabs, tabr, eidx)
    return _edge_mlp(edge_attr, gsum, w1e, w1g, global_attr,
                     b1.reshape(1, LATENT), W2, b2.reshape(1, D_OUT))


# speedup vs baseline: 6.6860x; 1.2732x over previous
import functools

import jax
import jax.numpy as jnp
from jax import lax
from jax.experimental import pallas as pl
from jax.experimental.pallas import tpu as pltpu
from jax.experimental.pallas import tpu_sc as plsc

N_NODES = 10000
N_EDGES = 320000
D_FEAT = 128
D_EDGE = 16
D_GLOBAL = 32
LATENT = 32
D_OUT = 128

_NC = 2
_NS = 16
_NW = _NC * _NS
_EPW = N_EDGES // _NW
_CH = 80
_NCHUNK = _EPW // _CH


def _proj_body(node_ref, w_ref, outs_ref, outr_ref):
    t = jnp.dot(node_ref[...], w_ref[...], preferred_element_type=jnp.float32)
    outs_ref[...] = t[:, :LATENT]
    outr_ref[...] = t[:, LATENT:]


def _node_projections(node_attr, w_sr):
    return pl.pallas_call(
        _proj_body,
        out_shape=[
            jax.ShapeDtypeStruct((N_NODES, LATENT), jnp.float32),
            jax.ShapeDtypeStruct((N_NODES, LATENT), jnp.float32),
        ],
    )(node_attr, w_sr)


_GROWS = _CH * LATENT // 128
_WROWS = _EPW * LATENT // 128


def _gather_body(tabs_hbm, tabr_hbm, eidx_hbm, out_hbm,
                 idxs_v, idxr_v, rs0, rs1, rr0, rr1, sv0, sv1,
                 g0, g1, s0, s1):
    rows_s, rows_r, sum_v = [rs0, rs1], [rr0, rr1], [sv0, sv1]
    gsem, ssem = [g0, g1], [s0, s1]
    wid = lax.axis_index("s") * _NC + lax.axis_index("c")
    base = pl.multiple_of(wid * _EPW, 8)
    pltpu.sync_copy(eidx_hbm.at[0, pl.ds(base, _EPW)], idxs_v)
    pltpu.sync_copy(eidx_hbm.at[1, pl.ds(base, _EPW)], idxr_v)

    def issue(j, b):
        cs = pl.ds(pl.multiple_of(j * _CH, 8), _CH)
        pltpu.async_copy(tabs_hbm.at[idxs_v.at[cs]], rows_s[b], gsem[b])
        pltpu.async_copy(tabr_hbm.at[idxr_v.at[cs]], rows_r[b], gsem[b])

    def finish(j, b):
        dr = pl.ds(0, _CH)
        pltpu.make_async_copy(tabs_hbm.at[idxs_v.at[dr]], rows_s[b],
                              gsem[b]).wait()
        pltpu.make_async_copy(tabr_hbm.at[idxr_v.at[dr]], rows_r[b],
                              gsem[b]).wait()
        pltpu.make_async_copy(out_hbm.at[pl.ds(0, _CH), pl.ds(0, LATENT)],
                              sum_v[b], ssem[b]).wait()
        for i in range(_CH * LATENT // 16):
            r, c = divmod(i, 2)
            sum_v[b][r, pl.ds(c * 16, 16)] = (
                rows_s[b][r, pl.ds(c * 16, 16)]
                + rows_r[b][r, pl.ds(c * 16, 16)])
        e0 = base + j * _CH
        blk = e0 // _BE
        rem = e0 - blk * _BE
        kk = rem // (_BE // 4)
        r0 = rem - kk * (_BE // 4)
        dst = out_hbm.at[pl.ds((_BE // 4) * blk + r0, _CH),
                         pl.ds(LATENT * kk, LATENT)]
        pltpu.async_copy(sum_v[b], dst, ssem[b])

    pltpu.async_copy(out_hbm.at[pl.ds(0, _CH), pl.ds(0, LATENT)],
                     sum_v[0], ssem[0])
    pltpu.async_copy(out_hbm.at[pl.ds(0, _CH), pl.ds(0, LATENT)],
                     sum_v[1], ssem[1])
    issue(0, 0)
    issue(1, 1)

    def pair(i, carry):
        finish(2 * i, 0)
        issue(2 * i + 2, 0)
        finish(2 * i + 1, 1)

        @pl.when(i < _NCHUNK // 2 - 1)
        def _():
            issue(2 * i + 3, 1)

        return carry

    lax.fori_loop(0, _NCHUNK // 2, pair, 0)
    finish(_NCHUNK - 1, 0)
    pltpu.make_async_copy(out_hbm.at[pl.ds(0, _CH), pl.ds(0, LATENT)],
                          sum_v[0], ssem[0]).wait()
    pltpu.make_async_copy(out_hbm.at[pl.ds(0, _CH), pl.ds(0, LATENT)],
                          sum_v[1], ssem[1]).wait()


def _gather_projections(tabs, tabr, edge_index):
    mesh = plsc.VectorSubcoreMesh(core_axis_name="c", subcore_axis_name="s")
    k = functools.partial(
        pl.kernel,
        out_type=jax.ShapeDtypeStruct((N_EDGES * LATENT // 128, 128),
                                      jnp.float32),
        mesh=mesh,
        scratch_types=(
            [pltpu.VMEM((_EPW,), jnp.int32)] * 2
            + [pltpu.VMEM((_CH, LATENT), jnp.float32)] * 6
            + [pltpu.SemaphoreType.DMA] * 4
        ),
        compiler_params=pltpu.CompilerParams(use_tc_tiling_on_sc=False),
    )(_gather_body)
    return k(tabs, tabr, edge_index)


_BE = 6400


def _mlp_body(e_ref, gsum_ref, w1e_ref, w1g_ref, g_ref, b1_ref,
              w2_ref, b2_ref, out_ref):
    bias = b1_ref[...] + jnp.dot(g_ref[...], w1g_ref[...],
                                 preferred_element_type=jnp.float32)
    gsum = jnp.concatenate(
        [gsum_ref[:, k * LATENT:(k + 1) * LATENT] for k in range(4)], axis=0)
    pre = (gsum
           + jnp.dot(e_ref[...], w1e_ref[...],
                     preferred_element_type=jnp.float32)
           + bias)
    h = jnp.maximum(pre, 0.0)
    out_ref[...] = jnp.dot(h, w2_ref[...],
                           preferred_element_type=jnp.float32) + b2_ref[...]


def _edge_mlp(edge_attr, gsum, w1e, w1g, g, b1, w2, b2):
    nblk = N_EDGES // _BE
    full = lambda shape: pl.BlockSpec(shape, lambda i: (0, 0))
    return pl.pallas_call(
        _mlp_body,
        grid=(nblk,),
        in_specs=[
            pl.BlockSpec((_BE, D_EDGE), lambda i: (i, 0)),
            pl.BlockSpec((_BE * LATENT // 128, 128), lambda i: (i, 0)),
            full((D_EDGE, LATENT)),
            full((D_GLOBAL, LATENT)),
            full((1, D_GLOBAL)),
            full((1, LATENT)),
            full((LATENT, D_OUT)),
            full((1, D_OUT)),
        ],
        out_specs=pl.BlockSpec((_BE, D_OUT), lambda i: (i, 0)),
        out_shape=jax.ShapeDtypeStruct((N_EDGES, D_OUT), jnp.float32),
    )(edge_attr, gsum, w1e, w1g, g, b1, w2, b2)


def kernel(node_attr, edge_index, edge_attr, global_attr, W1, b1, W2, b2):
    eidx = edge_index.astype(jnp.int32)
    w1e = W1[:D_EDGE]
    w_sr = W1[D_EDGE:D_EDGE + 2 * D_FEAT]
    w_sr = jnp.concatenate(
        [w_sr[:D_FEAT], w_sr[D_FEAT:]], axis=1)
    w1g = W1[D_EDGE + 2 * D_FEAT:]
    tabs, tabr = _node_projections(node_attr, w_sr)
    gsum = _gather_projections(tabs, tabr, eidx)
    return _edge_mlp(edge_attr, gsum, w1e, w1g, global_attr,
                     b1.reshape(1, LATENT), W2, b2.reshape(1, D_OUT))
